# baseline (speedup 1.0000x reference)
import functools

import jax
import jax.numpy as jnp
from jax import lax
from jax.experimental import pallas as pl
from jax.experimental.pallas import tpu as pltpu
from jax.experimental.pallas import tpu_sc as plsc

_B = 16384
_F = 26
_VOCAB = 100000
_DIM = 32

_NC = 2
_NS = 16
_NW = _NC * _NS
_ROWS_PER_W = _B // _NW
_CB = 4
_CHUNK_IDX = _CB * _F
_NCHUNKS = _ROWS_PER_W // _CB
_NBUF = 2


def _sc_body(idx_hbm, table_hbm, out_hbm, idx_v, buf0, buf1, out_v,
             sem0, sem1):
    wid = lax.axis_index("s") * _NC + lax.axis_index("c")

    pltpu.sync_copy(idx_hbm.at[wid], idx_v)

    bufs = (buf0, buf1)
    sems = (sem0, sem1)

    def start(chunk, k):
        pltpu.async_copy(table_hbm.at[idx_v.at[chunk]], bufs[k], sems[k])

    def wait(chunk, k):
        pltpu.make_async_copy(
            table_hbm.at[idx_v.at[chunk]], bufs[k], sems[k]).wait()

    def accum(chunk, k):
        buf = bufs[k]
        for lb in range(_CB):
            base = lb * _F
            acc0 = buf[base, pl.ds(0, 16)]
            acc1 = buf[base, pl.ds(16, 16)]
            for f in range(1, _F):
                acc0 = acc0 + buf[base + f, pl.ds(0, 16)]
                acc1 = acc1 + buf[base + f, pl.ds(16, 16)]
            row = chunk * _CB + lb
            out_v[row, pl.ds(0, 16)] = acc0
            out_v[row, pl.ds(16, 16)] = acc1

    for k in range(_NBUF):
        start(k, k)

    @pl.loop(0, _NCHUNKS - _NBUF, step=_NBUF)
    def _(c):
        for k in range(_NBUF):
            chunk = c + k
            wait(chunk, k)
            accum(chunk, k)
            start(chunk + _NBUF, k)

    for k in range(_NBUF):
        chunk = _NCHUNKS - _NBUF + k
        wait(chunk, k)
        accum(chunk, k)

    pltpu.sync_copy(out_v, out_hbm.at[pl.ds(wid * _ROWS_PER_W, _ROWS_PER_W)])


@jax.jit
def _multi_embed(idx_flat, table_flat):
    mesh = plsc.VectorSubcoreMesh(
        core_axis_name="c", subcore_axis_name="s",
        num_cores=_NC, num_subcores=_NS)
    run = pl.kernel(
        _sc_body,
        out_type=jax.ShapeDtypeStruct((_B, _DIM), jnp.float32),
        mesh=mesh,
        scratch_types=[
            pltpu.VMEM((_NCHUNKS, _CHUNK_IDX), jnp.int32),
            pltpu.VMEM((_CHUNK_IDX, _DIM), jnp.float32),
            pltpu.VMEM((_CHUNK_IDX, _DIM), jnp.float32),
            pltpu.VMEM((_ROWS_PER_W, _DIM), jnp.float32),
            pltpu.SemaphoreType.DMA,
            pltpu.SemaphoreType.DMA,
        ],
        compiler_params=pltpu.CompilerParams(use_tc_tiling_on_sc=False),
    )
    return run(idx_flat, table_flat)


def kernel(inputs, tables):
    table_flat = tables.reshape(_F * _VOCAB, _DIM)
    offs = (jnp.arange(_F, dtype=jnp.int32) * _VOCAB)[None, :]
    idx_flat = (inputs.astype(jnp.int32) + offs).reshape(
        _NW, _NCHUNKS, _CHUNK_IDX)
    return _multi_embed(idx_flat, table_flat)


# speedup vs baseline: 1.0013x; 1.0013x over previous
import functools

import jax
import jax.numpy as jnp
from jax import lax
from jax.experimental import pallas as pl
from jax.experimental.pallas import tpu as pltpu
from jax.experimental.pallas import tpu_sc as plsc

_B = 16384
_F = 26
_VOCAB = 100000
_DIM = 32

_NC = 2
_NS = 16
_NW = _NC * _NS
_ROWS_PER_W = _B // _NW
_CB = 4
_CHUNK_IDX = _CB * _F
_NCHUNKS = _ROWS_PER_W // _CB
_NBUF = 8


def _tree_sum(vals):
    while len(vals) > 1:
        vals = [a + b for a, b in zip(vals[::2], vals[1::2])] + (
            [vals[-1]] if len(vals) % 2 else [])
    return vals[0]


def _sc_body(idx_hbm, table_hbm, out_hbm, idx_v, bufs, out_v, sems):
    wid = lax.axis_index("s") * _NC + lax.axis_index("c")

    pltpu.sync_copy(idx_hbm.at[wid], idx_v)

    def start(chunk, k):
        pltpu.async_copy(table_hbm.at[idx_v.at[chunk]], bufs[k], sems[k])

    def wait(chunk, k):
        pltpu.make_async_copy(
            table_hbm.at[idx_v.at[chunk]], bufs[k], sems[k]).wait()

    def accum(chunk, k):
        buf = bufs[k]
        for lb in range(_CB):
            base = lb * _F
            acc0 = _tree_sum([buf[base + f, pl.ds(0, 16)]
                              for f in range(_F)])
            acc1 = _tree_sum([buf[base + f, pl.ds(16, 16)]
                              for f in range(_F)])
            row = chunk * _CB + lb
            out_v[row, pl.ds(0, 16)] = acc0
            out_v[row, pl.ds(16, 16)] = acc1

    for k in range(_NBUF):
        start(k, k)

    @pl.loop(0, _NCHUNKS - _NBUF, step=_NBUF)
    def _(c):
        for k in range(_NBUF):
            chunk = c + k
            wait(chunk, k)
            accum(chunk, k)
            start(chunk + _NBUF, k)

    for k in range(_NBUF):
        chunk = _NCHUNKS - _NBUF + k
        wait(chunk, k)
        accum(chunk, k)

    pltpu.sync_copy(out_v, out_hbm.at[pl.ds(wid * _ROWS_PER_W, _ROWS_PER_W)])


@jax.jit
def _multi_embed(idx_flat, table_flat):
    mesh = plsc.VectorSubcoreMesh(
        core_axis_name="c", subcore_axis_name="s",
        num_cores=_NC, num_subcores=_NS)
    run = pl.kernel(
        _sc_body,
        out_type=jax.ShapeDtypeStruct((_B, _DIM), jnp.float32),
        mesh=mesh,
        scratch_types=[
            pltpu.VMEM((_NCHUNKS, _CHUNK_IDX), jnp.int32),
            [pltpu.VMEM((_CHUNK_IDX, _DIM), jnp.float32)
             for _ in range(_NBUF)],
            pltpu.VMEM((_ROWS_PER_W, _DIM), jnp.float32),
            [pltpu.SemaphoreType.DMA for _ in range(_NBUF)],
        ],
        compiler_params=pltpu.CompilerParams(use_tc_tiling_on_sc=False),
    )
    return run(idx_flat, table_flat)


def kernel(inputs, tables):
    table_flat = tables.reshape(_F * _VOCAB, _DIM)
    offs = (jnp.arange(_F, dtype=jnp.int32) * _VOCAB)[None, :]
    idx_flat = (inputs.astype(jnp.int32) + offs).reshape(
        _NW, _NCHUNKS, _CHUNK_IDX)
    return _multi_embed(idx_flat, table_flat)
